# baseline (speedup 1.0000x reference)
import functools

import jax
import jax.numpy as jnp
from jax import lax
from jax.experimental import pallas as pl
from jax.experimental.pallas import tpu as pltpu
from jax.experimental.pallas import tpu_sc as plsc

N = 10000
E = 320000
D = 128
HID = 1024
OUT = 128
G = 64

NPAD = 10112
NC = 2
NS = 16
NW = NC * NS
EW = E // NW
C = 128
EWP = 10240
NCH = EWP // C
L16 = 16
PADV = -1e30


def _pre_body(x_ref, w_ref, asrc_ref, adst_ref, h_ref, as_ref, ad_ref,
              g_ref, exs_ref):
    h = jnp.dot(x_ref[...], w_ref[...], preferred_element_type=jnp.float32)
    h_ref[...] = h
    valid = lax.broadcasted_iota(jnp.int32, (NPAD,), 0) < N
    a_s = jnp.sum(h * asrc_ref[...], axis=1)
    a_d = jnp.sum(h * adst_ref[...], axis=1)
    a_s = jnp.where(valid, a_s, PADV)
    a_d = jnp.where(valid, a_d, PADV)
    as_ref[...] = a_s
    ad_ref[...] = a_d
    g = jnp.max(a_s) + jnp.max(a_d)
    g = jnp.where(g > 0, g, 0.2 * g)
    g_ref[...] = jnp.full((C,), g, jnp.float32)
    a = a_s + a_d
    a = jnp.where(a > 0, a, 0.2 * a)
    exs_ref[...] = jnp.exp(a - g)


_pre = pl.pallas_call(
    _pre_body,
    out_shape=(
        jax.ShapeDtypeStruct((NPAD, D), jnp.float32),
        jax.ShapeDtypeStruct((NPAD,), jnp.float32),
        jax.ShapeDtypeStruct((NPAD,), jnp.float32),
        jax.ShapeDtypeStruct((C,), jnp.float32),
        jax.ShapeDtypeStruct((NPAD,), jnp.float32),
    ),
)


def _sc_edge_body(h_hbm, as_hbm, ad_hbm, g_hbm, sd_hbm,
                  acc_hbm, den_hbm,
                  g_v, idxA, idxB, asvA, asvB, advA, advB, exA, exB,
                  rowsA, rowsB,
                  as_s, ad_s, acc_s, den_s,
                  semgA, semgB, semasA, semasB, semadA, semadB,
                  semrA, semrB, semeA, semeB, semdump):
    cid = lax.axis_index("c")
    sid = lax.axis_index("s")
    wid = cid * NS + sid

    pltpu.sync_copy(g_hbm, g_v)

    @pl.when(sid == 0)
    def _stage():
        pltpu.sync_copy(as_hbm, as_s)
        pltpu.sync_copy(ad_hbm, ad_s)

    zero = jnp.zeros((L16,), jnp.float32)

    def _zero_rows(i, _):
        for k in range(D // L16):
            rowsA[i, pl.ds(k * L16, L16)] = zero
        return 0

    lax.fori_loop(0, C, _zero_rows, 0)
    for k in range(C // L16):
        exA[pl.ds(k * L16, L16)] = zero
    for r in range(5):
        off = sid * 640 + r * C

        @pl.when(off < NPAD)
        def _z():
            pltpu.sync_copy(rowsA, acc_s.at[pl.ds(off, C)])
            pltpu.sync_copy(exA, den_s.at[pl.ds(off, C)])

    plsc.subcore_barrier()

    gv = g_v[pl.ds(0, L16)]

    def _fire(ci, idx, rows, asv, adv, semg, semas, semad):
        pltpu.sync_copy(sd_hbm.at[wid * NCH + ci], idx)
        sref = idx.at[0]
        dref = idx.at[1]
        pltpu.async_copy(h_hbm.at[sref], rows, semg)
        pltpu.async_copy(as_s.at[sref], asv, semas)
        pltpu.async_copy(ad_s.at[dref], adv, semad)

    def _process(idx, rows, asv, adv, ex,
                 semg, semas, semad, semr, seme):
        sref = idx.at[0]
        dref = idx.at[1]
        pltpu.make_async_copy(as_s.at[sref], asv, semas).wait()
        pltpu.make_async_copy(ad_s.at[dref], adv, semad).wait()
        for k in range(C // L16):
            a = asv[pl.ds(k * L16, L16)] + adv[pl.ds(k * L16, L16)]
            a = jnp.where(a > 0, a, 0.2 * a) - gv
            ex[pl.ds(k * L16, L16)] = jnp.exp(a)
        pltpu.make_async_copy(h_hbm.at[sref], rows, semg).wait()

        def _scale(i, _):
            spl = plsc.load_gather(ex, [jnp.full((L16,), i, jnp.int32)])
            for k in range(D // L16):
                rows[i, pl.ds(k * L16, L16)] = rows[i, pl.ds(k * L16, L16)] * spl
            return 0

        lax.fori_loop(0, C, _scale, 0)
        pltpu.async_copy(rows, acc_s.at[dref], semr, add=True)
        pltpu.async_copy(ex, den_s.at[dref], seme, add=True)

    def _drain(rows, ex, semr, seme):
        pltpu.make_async_copy(rows, acc_s.at[pl.ds(0, C)], semr).wait()
        pltpu.make_async_copy(ex, den_s.at[pl.ds(0, C)], seme).wait()

    _fire(0, idxA, rowsA, asvA, advA, semgA, semasA, semadA)

    def _pair(j, _):
        @pl.when(j >= 1)
        def _dB():
            _drain(rowsB, exB, semrB, semeB)

        _fire(2 * j + 1, idxB, rowsB, asvB, advB, semgB, semasB, semadB)
        _process(idxA, rowsA, asvA, advA, exA,
                 semgA, semasA, semadA, semrA, semeA)
        _process(idxB, rowsB, asvB, advB, exB,
                 semgB, semasB, semadB, semrB, semeB)
        _drain(rowsA, exA, semrA, semeA)

        @pl.when(2 * j + 2 < NCH)
        def _fA():
            _fire(2 * j + 2, idxA, rowsA, asvA, advA, semgA, semasA, semadA)

        return 0

    lax.fori_loop(0, NCH // 2, _pair, 0)
    _drain(rowsB, exB, semrB, semeB)

    plsc.subcore_barrier()

    for r in range(5):
        off = sid * 640 + r * C

        @pl.when(off < NPAD)
        def _dump():
            pltpu.async_copy(acc_s.at[pl.ds(off, C)], rowsA, semdump).wait()
            pltpu.async_copy(rowsA, acc_hbm.at[cid].at[pl.ds(off, C)],
                             semdump).wait()
            pltpu.async_copy(den_s.at[pl.ds(off, C)], exA, semdump).wait()
            pltpu.async_copy(exA, den_hbm.at[cid].at[pl.ds(off, C)],
                             semdump).wait()


_sc_edge = pl.kernel(
    _sc_edge_body,
    out_type=(
        jax.ShapeDtypeStruct((NC, NPAD, D), jnp.float32),
        jax.ShapeDtypeStruct((NC, NPAD), jnp.float32),
    ),
    mesh=plsc.VectorSubcoreMesh(core_axis_name="c", subcore_axis_name="s",
                                num_cores=NC, num_subcores=NS),
    compiler_params=pltpu.CompilerParams(needs_layout_passes=False),
    scratch_types=[
        pltpu.VMEM((C,), jnp.float32),
        pltpu.VMEM((2, C), jnp.int32),
        pltpu.VMEM((2, C), jnp.int32),
        pltpu.VMEM((C,), jnp.float32),
        pltpu.VMEM((C,), jnp.float32),
        pltpu.VMEM((C,), jnp.float32),
        pltpu.VMEM((C,), jnp.float32),
        pltpu.VMEM((C,), jnp.float32),
        pltpu.VMEM((C,), jnp.float32),
        pltpu.VMEM((C, D), jnp.float32),
        pltpu.VMEM((C, D), jnp.float32),
        pltpu.VMEM_SHARED((NPAD,), jnp.float32),
        pltpu.VMEM_SHARED((NPAD,), jnp.float32),
        pltpu.VMEM_SHARED((NPAD, D), jnp.float32),
        pltpu.VMEM_SHARED((NPAD,), jnp.float32),
        pltpu.SemaphoreType.DMA,
        pltpu.SemaphoreType.DMA,
        pltpu.SemaphoreType.DMA,
        pltpu.SemaphoreType.DMA,
        pltpu.SemaphoreType.DMA,
        pltpu.SemaphoreType.DMA,
        pltpu.SemaphoreType.DMA,
        pltpu.SemaphoreType.DMA,
        pltpu.SemaphoreType.DMA,
        pltpu.SemaphoreType.DMA,
        pltpu.SemaphoreType.DMA,
    ],
)


def _combine_body(acc_ref, den_ref, h_ref, exs_ref, b_ref, o_ref, *, relu):
    exs = exs_ref[...]
    num = acc_ref[0] + acc_ref[1] + exs[:, None] * h_ref[...]
    den = den_ref[0] + den_ref[1] + exs + 1e-16
    o = num / den[:, None] + b_ref[...]
    if relu:
        o = jnp.maximum(o, 0.0)
    o_ref[...] = o


def _make_combine(relu):
    return pl.pallas_call(
        functools.partial(_combine_body, relu=relu),
        out_shape=jax.ShapeDtypeStruct((NPAD, D), jnp.float32),
    )


_combine = _make_combine(False)
_combine_relu = _make_combine(True)


def _pool_mlp_body(x_ref, batch_ref, w1_ref, b1_ref, w2_ref, b2_ref, o_ref):
    gids = lax.broadcasted_iota(jnp.int32, (G, NPAD), 0)
    onehot = (batch_ref[...][None, :] == gids).astype(jnp.float32)
    sums = jnp.dot(onehot, x_ref[...], preferred_element_type=jnp.float32)
    cnt = jnp.sum(onehot, axis=1)
    pooled = sums / jnp.maximum(cnt, 1.0)[:, None]
    r1 = jnp.dot(jnp.maximum(pooled, 0.0), w1_ref[...],
                 preferred_element_type=jnp.float32) + b1_ref[...]
    r2 = jnp.dot(jnp.maximum(r1, 0.0), w2_ref[...],
                 preferred_element_type=jnp.float32) + b2_ref[...]
    o_ref[...] = r2


_pool_mlp = pl.pallas_call(
    _pool_mlp_body,
    out_shape=jax.ShapeDtypeStruct((G, OUT), jnp.float32),
)


def kernel(x, edge_index, edge_weight, batch,
           W0, att_src0, att_dst0, bias0,
           W1, att_src1, att_dst1, bias1,
           W2, att_src2, att_dst2, bias2,
           fc1_w, fc1_b, fc2_w, fc2_b):
    s = jnp.pad(edge_index[0].astype(jnp.int32).reshape(NW, EW),
                ((0, 0), (0, EWP - EW)), constant_values=N)
    d = jnp.pad(edge_index[1].astype(jnp.int32).reshape(NW, EW),
                ((0, 0), (0, EWP - EW)), constant_values=N)
    sd = jnp.stack([s.reshape(NW, NCH, C), d.reshape(NW, NCH, C)],
                   axis=2).reshape(NW * NCH, 2, C)
    batchp = jnp.pad(batch.astype(jnp.int32), (0, NPAD - N),
                     constant_values=G)
    xp = jnp.pad(x, ((0, NPAD - N), (0, 0)))

    layers = [
        (W0, att_src0, att_dst0, bias0, False),
        (W1, att_src1, att_dst1, bias1, True),
        (W2, att_src2, att_dst2, bias2, True),
    ]
    for W, a_s, a_d, b, relu in layers:
        h, asrc, adst, g, exs = _pre(xp, W, a_s[None, :], a_d[None, :])
        acc, den = _sc_edge(h, asrc, adst, g, sd)
        comb = _combine_relu if relu else _combine
        xp = comb(acc, den, h, exs, b[None, :])

    return _pool_mlp(xp, batchp, fc1_w, fc1_b[None, :], fc2_w, fc2_b[None, :])


# speedup vs baseline: 1.4706x; 1.4706x over previous
import functools

import jax
import jax.numpy as jnp
from jax import lax
from jax.experimental import pallas as pl
from jax.experimental.pallas import tpu as pltpu
from jax.experimental.pallas import tpu_sc as plsc

N = 10000
E = 320000
D = 128
HID = 1024
OUT = 128
G = 64

NPAD = 10112
NC = 2
NS = 16
NW = NC * NS
EW = E // NW
C = 128
EWP = 10112
NCH = EWP // C
L16 = 16
PADV = -1e30


def _pre_body(x_ref, w_ref, asrc_ref, adst_ref, h_ref, as_ref, ad_ref,
              g_ref, exs_ref):
    h = jnp.dot(x_ref[...], w_ref[...], preferred_element_type=jnp.float32)
    h_ref[...] = h
    valid = lax.broadcasted_iota(jnp.int32, (NPAD,), 0) < N
    a_s = jnp.sum(h * asrc_ref[...], axis=1)
    a_d = jnp.sum(h * adst_ref[...], axis=1)
    a_s = jnp.where(valid, a_s, PADV)
    a_d = jnp.where(valid, a_d, PADV)
    as_ref[...] = a_s
    ad_ref[...] = a_d
    g = jnp.max(a_s) + jnp.max(a_d)
    g = jnp.where(g > 0, g, 0.2 * g)
    g_ref[...] = jnp.full((C,), g, jnp.float32)
    a = a_s + a_d
    a = jnp.where(a > 0, a, 0.2 * a)
    exs_ref[...] = jnp.exp(a - g)


_pre = pl.pallas_call(
    _pre_body,
    out_shape=(
        jax.ShapeDtypeStruct((NPAD, D), jnp.float32),
        jax.ShapeDtypeStruct((NPAD,), jnp.float32),
        jax.ShapeDtypeStruct((NPAD,), jnp.float32),
        jax.ShapeDtypeStruct((C,), jnp.float32),
        jax.ShapeDtypeStruct((NPAD,), jnp.float32),
    ),
)


def _sc_edge_body(h_hbm, as_hbm, ad_hbm, g_hbm, sd_hbm,
                  acc_hbm, den_hbm,
                  g_v, sdA, sdB, asvA, asvB, advA, advB, exA, exB,
                  rowsA, rowsB,
                  as_s, ad_s, acc_s, den_s,
                  semgA, semgB, semasA, semasB, semadA, semadB,
                  semrA, semrB, semeA, semeB, semdump):
    cid = lax.axis_index("c")
    sid = lax.axis_index("s")
    wid = cid * NS + sid

    pltpu.sync_copy(g_hbm, g_v)

    @pl.when(sid == 0)
    def _stage():
        pltpu.sync_copy(as_hbm, as_s)
        pltpu.sync_copy(ad_hbm, ad_s)

    zero = jnp.zeros((L16,), jnp.float32)

    def _zero_rows(i, _):
        for k in range(D // L16):
            rowsA[i, pl.ds(k * L16, L16)] = zero
        return 0

    lax.fori_loop(0, C, _zero_rows, 0)
    for k in range(C // L16):
        exA[pl.ds(k * L16, L16)] = zero
    for r in range(5):
        off = sid * 640 + r * C

        @pl.when(off < NPAD)
        def _z():
            pltpu.sync_copy(rowsA, acc_s.at[pl.ds(off, C)])
            pltpu.sync_copy(exA, den_s.at[pl.ds(off, C)])

    plsc.subcore_barrier()

    gv = g_v[pl.ds(0, L16)]

    def _fire(ci, sd, rows, asv, adv, semg, semas, semad):
        pltpu.sync_copy(sd_hbm.at[wid * NCH + ci], sd)
        sref = sd.at[0]
        dref = sd.at[1]
        dg = pltpu.async_copy(h_hbm.at[sref], rows, semg)
        da = pltpu.async_copy(as_s.at[sref], asv, semas)
        dd = pltpu.async_copy(ad_s.at[dref], adv, semad)
        return dg, da, dd

    def _process(descs, sd, rows, asv, adv, ex, semr, seme):
        dg, da, dd = descs
        da.wait()
        dd.wait()
        for k in range(C // L16):
            a = asv[pl.ds(k * L16, L16)] + adv[pl.ds(k * L16, L16)]
            a = jnp.where(a > 0, a, 0.2 * a) - gv
            ex[pl.ds(k * L16, L16)] = jnp.exp(a)
        dg.wait()

        def _scale(i, _):
            spl = plsc.load_gather(ex, [jnp.full((L16,), i, jnp.int32)])
            for k in range(D // L16):
                rows[i, pl.ds(k * L16, L16)] = rows[i, pl.ds(k * L16, L16)] * spl
            return 0

        lax.fori_loop(0, C, _scale, 0, unroll=4)
        pltpu.async_copy(rows, acc_s.at[sd.at[1]], semr, add=True)
        pltpu.async_copy(ex, den_s.at[sd.at[1]], seme, add=True)

    def _drain(rows, ex, semr, seme):
        pltpu.make_async_copy(rows, acc_s.at[pl.ds(0, C)], semr).wait()
        pltpu.make_async_copy(ex, den_s.at[pl.ds(0, C)], seme).wait()

    def _pair(i, _):
        ca = 2 * i
        cb = 2 * i + 1

        @pl.when(i > 0)
        def _dA():
            _drain(rowsA, exA, semrA, semeA)

        descA = _fire(ca, sdA, rowsA, asvA, advA, semgA, semasA, semadA)

        @pl.when(i > 0)
        def _dB():
            _drain(rowsB, exB, semrB, semeB)

        descB = _fire(cb, sdB, rowsB, asvB, advB, semgB, semasB, semadB)
        _process(descA, sdA, rowsA, asvA, advA, exA, semrA, semeA)
        _process(descB, sdB, rowsB, asvB, advB, exB, semrB, semeB)
        return 0

    lax.fori_loop(0, NCH // 2, _pair, 0)
    _drain(rowsA, exA, semrA, semeA)
    descA = _fire(NCH - 1, sdA, rowsA, asvA, advA, semgA, semasA, semadA)
    _drain(rowsB, exB, semrB, semeB)
    _process(descA, sdA, rowsA, asvA, advA, exA, semrA, semeA)
    _drain(rowsA, exA, semrA, semeA)

    plsc.subcore_barrier()

    for r in range(5):
        off = sid * 640 + r * C

        @pl.when(off < NPAD)
        def _dump():
            pltpu.async_copy(acc_s.at[pl.ds(off, C)], rowsA, semdump).wait()
            pltpu.async_copy(rowsA, acc_hbm.at[cid].at[pl.ds(off, C)],
                             semdump).wait()
            pltpu.async_copy(den_s.at[pl.ds(off, C)], exA, semdump).wait()
            pltpu.async_copy(exA, den_hbm.at[cid].at[pl.ds(off, C)],
                             semdump).wait()


_sc_edge = pl.kernel(
    _sc_edge_body,
    out_type=(
        jax.ShapeDtypeStruct((NC, NPAD, D), jnp.float32),
        jax.ShapeDtypeStruct((NC, NPAD), jnp.float32),
    ),
    mesh=plsc.VectorSubcoreMesh(core_axis_name="c", subcore_axis_name="s",
                                num_cores=NC, num_subcores=NS),
    compiler_params=pltpu.CompilerParams(needs_layout_passes=False),
    scratch_types=[
        pltpu.VMEM((C,), jnp.float32),
        pltpu.VMEM((2, C), jnp.int32),
        pltpu.VMEM((2, C), jnp.int32),
        pltpu.VMEM((C,), jnp.float32),
        pltpu.VMEM((C,), jnp.float32),
        pltpu.VMEM((C,), jnp.float32),
        pltpu.VMEM((C,), jnp.float32),
        pltpu.VMEM((C,), jnp.float32),
        pltpu.VMEM((C,), jnp.float32),
        pltpu.VMEM((C, D), jnp.float32),
        pltpu.VMEM((C, D), jnp.float32),
        pltpu.VMEM_SHARED((NPAD,), jnp.float32),
        pltpu.VMEM_SHARED((NPAD,), jnp.float32),
        pltpu.VMEM_SHARED((NPAD, D), jnp.float32),
        pltpu.VMEM_SHARED((NPAD,), jnp.float32),
        pltpu.SemaphoreType.DMA,
        pltpu.SemaphoreType.DMA,
        pltpu.SemaphoreType.DMA,
        pltpu.SemaphoreType.DMA,
        pltpu.SemaphoreType.DMA,
        pltpu.SemaphoreType.DMA,
        pltpu.SemaphoreType.DMA,
        pltpu.SemaphoreType.DMA,
        pltpu.SemaphoreType.DMA,
        pltpu.SemaphoreType.DMA,
        pltpu.SemaphoreType.DMA,
    ],
)


def _combine_body(acc_ref, den_ref, h_ref, exs_ref, b_ref, o_ref, *, relu):
    exs = exs_ref[...]
    num = acc_ref[0] + acc_ref[1] + exs[:, None] * h_ref[...]
    den = den_ref[0] + den_ref[1] + exs + 1e-16
    o = num / den[:, None] + b_ref[...]
    if relu:
        o = jnp.maximum(o, 0.0)
    o_ref[...] = o


def _make_combine(relu):
    return pl.pallas_call(
        functools.partial(_combine_body, relu=relu),
        out_shape=jax.ShapeDtypeStruct((NPAD, D), jnp.float32),
    )


_combine = _make_combine(False)
_combine_relu = _make_combine(True)


def _pool_mlp_body(x_ref, batch_ref, w1_ref, b1_ref, w2_ref, b2_ref, o_ref):
    gids = lax.broadcasted_iota(jnp.int32, (G, NPAD), 0)
    onehot = (batch_ref[...][None, :] == gids).astype(jnp.float32)
    sums = jnp.dot(onehot, x_ref[...], preferred_element_type=jnp.float32)
    cnt = jnp.sum(onehot, axis=1)
    pooled = sums / jnp.maximum(cnt, 1.0)[:, None]
    r1 = jnp.dot(jnp.maximum(pooled, 0.0), w1_ref[...],
                 preferred_element_type=jnp.float32) + b1_ref[...]
    r2 = jnp.dot(jnp.maximum(r1, 0.0), w2_ref[...],
                 preferred_element_type=jnp.float32) + b2_ref[...]
    o_ref[...] = r2


_pool_mlp = pl.pallas_call(
    _pool_mlp_body,
    out_shape=jax.ShapeDtypeStruct((G, OUT), jnp.float32),
)


def kernel(x, edge_index, edge_weight, batch,
           W0, att_src0, att_dst0, bias0,
           W1, att_src1, att_dst1, bias1,
           W2, att_src2, att_dst2, bias2,
           fc1_w, fc1_b, fc2_w, fc2_b):
    s = jnp.pad(edge_index[0].astype(jnp.int32).reshape(NW, EW),
                ((0, 0), (0, EWP - EW)), constant_values=N)
    d = jnp.pad(edge_index[1].astype(jnp.int32).reshape(NW, EW),
                ((0, 0), (0, EWP - EW)), constant_values=N)
    sd = jnp.stack([s.reshape(NW, NCH, C), d.reshape(NW, NCH, C)],
                   axis=2).reshape(NW * NCH, 2, C)
    batchp = jnp.pad(batch.astype(jnp.int32), (0, NPAD - N),
                     constant_values=G)
    xp = jnp.pad(x, ((0, NPAD - N), (0, 0)))

    layers = [
        (W0, att_src0, att_dst0, bias0, False),
        (W1, att_src1, att_dst1, bias1, True),
        (W2, att_src2, att_dst2, bias2, True),
    ]
    for W, a_s, a_d, b, relu in layers:
        h, asrc, adst, g, exs = _pre(xp, W, a_s[None, :], a_d[None, :])
        acc, den = _sc_edge(h, asrc, adst, g, sd)
        comb = _combine_relu if relu else _combine
        xp = comb(acc, den, h, exs, b[None, :])

    return _pool_mlp(xp, batchp, fc1_w, fc1_b[None, :], fc2_w, fc2_b[None, :])


# speedup vs baseline: 1.4922x; 1.0147x over previous
import functools

import jax
import jax.numpy as jnp
from jax import lax
from jax.experimental import pallas as pl
from jax.experimental.pallas import tpu as pltpu
from jax.experimental.pallas import tpu_sc as plsc

N = 10000
E = 320000
D = 128
HID = 1024
OUT = 128
G = 64

NPAD = 10112
NC = 2
NS = 16
NW = NC * NS
EW = E // NW
C = 128
EWP = 10112
NCH = EWP // C
L16 = 16
PADV = -1e30


def _pre_body(x_ref, w_ref, asrc_ref, adst_ref, h_ref, as_ref, ad_ref,
              g_ref, exs_ref):
    h = jnp.dot(x_ref[...], w_ref[...], preferred_element_type=jnp.float32)
    h_ref[...] = h
    valid = lax.broadcasted_iota(jnp.int32, (NPAD,), 0) < N
    a_s = jnp.sum(h * asrc_ref[...], axis=1)
    a_d = jnp.sum(h * adst_ref[...], axis=1)
    a_s = jnp.where(valid, a_s, PADV)
    a_d = jnp.where(valid, a_d, PADV)
    as_ref[...] = a_s
    ad_ref[...] = a_d
    g = jnp.max(a_s) + jnp.max(a_d)
    g = jnp.where(g > 0, g, 0.2 * g)
    g_ref[...] = jnp.full((C,), g, jnp.float32)
    a = a_s + a_d
    a = jnp.where(a > 0, a, 0.2 * a)
    exs_ref[...] = jnp.exp(a - g)


_pre = pl.pallas_call(
    _pre_body,
    out_shape=(
        jax.ShapeDtypeStruct((NPAD, D), jnp.float32),
        jax.ShapeDtypeStruct((NPAD,), jnp.float32),
        jax.ShapeDtypeStruct((NPAD,), jnp.float32),
        jax.ShapeDtypeStruct((C,), jnp.float32),
        jax.ShapeDtypeStruct((NPAD,), jnp.float32),
    ),
)


def _sc_edge_body(h_hbm, as_hbm, ad_hbm, g_hbm, sd_hbm,
                  acc_hbm, den_hbm,
                  g_v, sdA, sdB, asvA, asvB, advA, advB, exA, exB,
                  rowsA, rowsB,
                  as_s, ad_s, acc_s, den_s,
                  semgA, semgB, semasA, semasB, semadA, semadB,
                  semrA, semrB, semeA, semeB, semdump):
    cid = lax.axis_index("c")
    sid = lax.axis_index("s")
    wid = cid * NS + sid

    pltpu.sync_copy(g_hbm, g_v)

    @pl.when(sid == 0)
    def _stage():
        pltpu.sync_copy(as_hbm, as_s)
        pltpu.sync_copy(ad_hbm, ad_s)

    zero = jnp.zeros((L16,), jnp.float32)

    def _zero_rows(i, _):
        for k in range(D // L16):
            rowsA[i, pl.ds(k * L16, L16)] = zero
        return 0

    lax.fori_loop(0, C, _zero_rows, 0)
    for k in range(C // L16):
        exA[pl.ds(k * L16, L16)] = zero
    for r in range(5):
        off = sid * 640 + r * C

        @pl.when(off < NPAD)
        def _z():
            pltpu.sync_copy(rowsA, acc_s.at[pl.ds(off, C)])
            pltpu.sync_copy(exA, den_s.at[pl.ds(off, C)])

    plsc.subcore_barrier()

    gv = g_v[pl.ds(0, L16)]

    def _fire(ci, sd, rows, asv, adv, semg, semas, semad):
        pltpu.sync_copy(sd_hbm.at[wid * NCH + ci], sd)
        sref = sd.at[0]
        dref = sd.at[1]
        dg = pltpu.async_copy(h_hbm.at[sref], rows, semg)
        da = pltpu.async_copy(as_s.at[sref], asv, semas)
        dd = pltpu.async_copy(ad_s.at[dref], adv, semad)
        return dg, da, dd

    def _process(descs, sd, rows, asv, adv, ex, semr, seme):
        dg, da, dd = descs
        da.wait()
        dd.wait()
        for k in range(C // L16):
            a = asv[pl.ds(k * L16, L16)] + adv[pl.ds(k * L16, L16)]
            a = jnp.where(a > 0, a, 0.2 * a) - gv
            ex[pl.ds(k * L16, L16)] = jnp.exp(a)
        dg.wait()

        def _scale(i, _):
            spl = plsc.load_gather(ex, [jnp.full((L16,), i, jnp.int32)])
            for k in range(D // L16):
                rows[i, pl.ds(k * L16, L16)] = rows[i, pl.ds(k * L16, L16)] * spl
            return 0

        lax.fori_loop(0, C, _scale, 0, unroll=4)
        pltpu.async_copy(rows, acc_s.at[sd.at[1]], semr, add=True)
        pltpu.async_copy(ex, den_s.at[sd.at[1]], seme, add=True)

    def _drain(rows, ex, semr, seme):
        pltpu.make_async_copy(rows, acc_s.at[pl.ds(0, C)], semr).wait()
        pltpu.make_async_copy(ex, den_s.at[pl.ds(0, C)], seme).wait()

    def _pair(i, _):
        ca = 2 * i
        cb = 2 * i + 1

        @pl.when(i > 0)
        def _dA():
            _drain(rowsA, exA, semrA, semeA)

        descA = _fire(ca, sdA, rowsA, asvA, advA, semgA, semasA, semadA)

        @pl.when(i > 0)
        def _dB():
            _drain(rowsB, exB, semrB, semeB)

        descB = _fire(cb, sdB, rowsB, asvB, advB, semgB, semasB, semadB)
        _process(descA, sdA, rowsA, asvA, advA, exA, semrA, semeA)
        _process(descB, sdB, rowsB, asvB, advB, exB, semrB, semeB)
        return 0

    lax.fori_loop(0, NCH // 2, _pair, 0)
    _drain(rowsA, exA, semrA, semeA)
    descA = _fire(NCH - 1, sdA, rowsA, asvA, advA, semgA, semasA, semadA)
    _drain(rowsB, exB, semrB, semeB)
    _process(descA, sdA, rowsA, asvA, advA, exA, semrA, semeA)
    _drain(rowsA, exA, semrA, semeA)

    plsc.subcore_barrier()

    for r in range(5):
        off = sid * 640 + r * C

        @pl.when(off < NPAD)
        def _dump():
            pltpu.async_copy(acc_s.at[pl.ds(off, C)], rowsA, semdump).wait()
            pltpu.async_copy(rowsA, acc_hbm.at[cid].at[pl.ds(off, C)],
                             semdump).wait()
            pltpu.async_copy(den_s.at[pl.ds(off, C)], exA, semdump).wait()
            pltpu.async_copy(exA, den_hbm.at[cid].at[pl.ds(off, C)],
                             semdump).wait()


_sc_edge = pl.kernel(
    _sc_edge_body,
    out_type=(
        jax.ShapeDtypeStruct((NC, NPAD, D), jnp.float32),
        jax.ShapeDtypeStruct((NC, NPAD), jnp.float32),
    ),
    mesh=plsc.VectorSubcoreMesh(core_axis_name="c", subcore_axis_name="s",
                                num_cores=NC, num_subcores=NS),
    compiler_params=pltpu.CompilerParams(needs_layout_passes=False),
    scratch_types=[
        pltpu.VMEM((C,), jnp.float32),
        pltpu.VMEM((2, C), jnp.int32),
        pltpu.VMEM((2, C), jnp.int32),
        pltpu.VMEM((C,), jnp.float32),
        pltpu.VMEM((C,), jnp.float32),
        pltpu.VMEM((C,), jnp.float32),
        pltpu.VMEM((C,), jnp.float32),
        pltpu.VMEM((C,), jnp.float32),
        pltpu.VMEM((C,), jnp.float32),
        pltpu.VMEM((C, D), jnp.float32),
        pltpu.VMEM((C, D), jnp.float32),
        pltpu.VMEM_SHARED((NPAD,), jnp.float32),
        pltpu.VMEM_SHARED((NPAD,), jnp.float32),
        pltpu.VMEM_SHARED((NPAD, D), jnp.float32),
        pltpu.VMEM_SHARED((NPAD,), jnp.float32),
        pltpu.SemaphoreType.DMA,
        pltpu.SemaphoreType.DMA,
        pltpu.SemaphoreType.DMA,
        pltpu.SemaphoreType.DMA,
        pltpu.SemaphoreType.DMA,
        pltpu.SemaphoreType.DMA,
        pltpu.SemaphoreType.DMA,
        pltpu.SemaphoreType.DMA,
        pltpu.SemaphoreType.DMA,
        pltpu.SemaphoreType.DMA,
        pltpu.SemaphoreType.DMA,
    ],
)


def _combine(acc_ref, den_ref, h_ref, exs_ref, b_ref, relu):
    exs = exs_ref[...]
    num = acc_ref[0] + acc_ref[1] + exs[:, None] * h_ref[...]
    den = den_ref[0] + den_ref[1] + exs + 1e-16
    o = num / den[:, None] + b_ref[...]
    if relu:
        o = jnp.maximum(o, 0.0)
    return o


def _mid_body(acc_ref, den_ref, hp_ref, exsp_ref, b_ref,
              w_ref, asrc_ref, adst_ref,
              h_ref, as_ref, ad_ref, g_ref, exs_ref, *, relu):
    x = _combine(acc_ref, den_ref, hp_ref, exsp_ref, b_ref, relu)
    h = jnp.dot(x, w_ref[...], preferred_element_type=jnp.float32)
    h_ref[...] = h
    valid = lax.broadcasted_iota(jnp.int32, (NPAD,), 0) < N
    a_s = jnp.sum(h * asrc_ref[...], axis=1)
    a_d = jnp.sum(h * adst_ref[...], axis=1)
    a_s = jnp.where(valid, a_s, PADV)
    a_d = jnp.where(valid, a_d, PADV)
    as_ref[...] = a_s
    ad_ref[...] = a_d
    g = jnp.max(a_s) + jnp.max(a_d)
    g = jnp.where(g > 0, g, 0.2 * g)
    g_ref[...] = jnp.full((C,), g, jnp.float32)
    a = a_s + a_d
    a = jnp.where(a > 0, a, 0.2 * a)
    exs_ref[...] = jnp.exp(a - g)


def _make_mid(relu):
    return pl.pallas_call(
        functools.partial(_mid_body, relu=relu),
        out_shape=(
            jax.ShapeDtypeStruct((NPAD, D), jnp.float32),
            jax.ShapeDtypeStruct((NPAD,), jnp.float32),
            jax.ShapeDtypeStruct((NPAD,), jnp.float32),
            jax.ShapeDtypeStruct((C,), jnp.float32),
            jax.ShapeDtypeStruct((NPAD,), jnp.float32),
        ),
    )


_mid_norelu = _make_mid(False)
_mid_relu = _make_mid(True)


def _pool_mlp_body(acc_ref, den_ref, hp_ref, exsp_ref, b_ref, batch_ref,
                   w1_ref, b1_ref, w2_ref, b2_ref, o_ref):
    x = _combine(acc_ref, den_ref, hp_ref, exsp_ref, b_ref, True)
    gids = lax.broadcasted_iota(jnp.int32, (G, NPAD), 0)
    onehot = (batch_ref[...][None, :] == gids).astype(jnp.float32)
    sums = jnp.dot(onehot, x, preferred_element_type=jnp.float32)
    cnt = jnp.sum(onehot, axis=1)
    pooled = sums / jnp.maximum(cnt, 1.0)[:, None]
    r1 = jnp.dot(jnp.maximum(pooled, 0.0), w1_ref[...],
                 preferred_element_type=jnp.float32) + b1_ref[...]
    r2 = jnp.dot(jnp.maximum(r1, 0.0), w2_ref[...],
                 preferred_element_type=jnp.float32) + b2_ref[...]
    o_ref[...] = r2


_pool_mlp = pl.pallas_call(
    _pool_mlp_body,
    out_shape=jax.ShapeDtypeStruct((G, OUT), jnp.float32),
)


def kernel(x, edge_index, edge_weight, batch,
           W0, att_src0, att_dst0, bias0,
           W1, att_src1, att_dst1, bias1,
           W2, att_src2, att_dst2, bias2,
           fc1_w, fc1_b, fc2_w, fc2_b):
    s = jnp.pad(edge_index[0].astype(jnp.int32).reshape(NW, EW),
                ((0, 0), (0, EWP - EW)), constant_values=N)
    d = jnp.pad(edge_index[1].astype(jnp.int32).reshape(NW, EW),
                ((0, 0), (0, EWP - EW)), constant_values=N)
    sd = jnp.stack([s.reshape(NW, NCH, C), d.reshape(NW, NCH, C)],
                   axis=2).reshape(NW * NCH, 2, C)
    batchp = jnp.pad(batch.astype(jnp.int32), (0, NPAD - N),
                     constant_values=G)
    xp = jnp.pad(x, ((0, NPAD - N), (0, 0)))

    h, asrc, adst, g1, exs1 = _pre(xp, W0, att_src0[None, :],
                                   att_dst0[None, :])
    acc, den = _sc_edge(h, asrc, adst, g1, sd)
    h, asrc, adst, g2, exs2 = _mid_norelu(
        acc, den, h, exs1, bias0[None, :], W1,
        att_src1[None, :], att_dst1[None, :])
    acc, den = _sc_edge(h, asrc, adst, g2, sd)
    h, asrc, adst, g3, exs3 = _mid_relu(
        acc, den, h, exs2, bias1[None, :], W2,
        att_src2[None, :], att_dst2[None, :])
    acc, den = _sc_edge(h, asrc, adst, g3, sd)
    return _pool_mlp(acc, den, h, exs3, bias2[None, :], batchp,
                     fc1_w, fc1_b[None, :], fc2_w, fc2_b[None, :])
